# speedup vs baseline: 10.9459x; 10.9459x over previous
import numpy as np
import jax
import jax.numpy as jnp
from jax.experimental import pallas as pl

B, N, D_IN, H = 64, 128, 4, 64


def _fused_kernel(x_ref, w1_ref, b1_ref, w2_ref, b2_ref, rw_ref, rb_ref,
                  ow1_ref, ob1_ref, ow2_ref, ob2_ref, ow3_ref, ob3_ref,
                  o_ref):
    x = x_ref[0]
    px, py = x[:, 0:1], x[:, 1:2]
    vx, vy = x[:, 2:3], x[:, 3:4]
    theta = jnp.arctan2(vy, vx)
    c, s = jnp.cos(theta), jnp.sin(theta)
    cvx = c * vx + s * vy
    cvy = c * vy - s * vx
    node = jnp.concatenate([px, py, c, s, cvx, cvy], axis=1)
    nodeT = node.T
    pxr, pyr = nodeT[0:1, :], nodeT[1:2, :]
    cr, sr = nodeT[2:3, :], nodeT[3:4, :]
    cvxr, cvyr = nodeT[4:5, :], nodeT[5:6, :]

    dx = px - pxr
    dy = py - pyr
    rrx = cr * dx + sr * dy
    rry = cr * dy - sr * dx
    euler = jnp.arctan2(s * cr - c * sr, c * cr + s * sr) * np.float32(1.0 / np.pi)
    dist = jnp.sqrt(dx * dx + dy * dy)
    sph = jnp.arctan2(rry, rrx)
    rvx = cr * vx + sr * vy
    rvy = cr * vy - sr * vx
    zero = jnp.zeros_like(dx)
    cvxb = zero + cvxr
    cvyb = zero + cvyr

    feats = jnp.stack(
        [rrx, rry, euler, dist, sph, rvx, rvy, zero, zero, cvxb, cvyb],
        axis=-1)
    e = feats.reshape(N * N, 11)
    h = jnp.dot(e, w1_ref[:], preferred_element_type=jnp.float32) + b1_ref[:]
    h = h * jax.nn.sigmoid(h)
    msg = jnp.dot(h, w2_ref[:], preferred_element_type=jnp.float32) + b2_ref[:]

    iota_s = jax.lax.broadcasted_iota(jnp.int32, (N, N), 0)
    iota_r = jax.lax.broadcasted_iota(jnp.int32, (N, N), 1)
    mask = (iota_s != iota_r).astype(jnp.float32)
    msg3 = msg.reshape(N, N, H) * mask[:, :, None]
    agg = jnp.sum(msg3, axis=0) * np.float32(1.0 / (N - 1))

    rel_feat = jnp.concatenate(
        [jnp.zeros_like(cvx), jnp.zeros_like(cvx), cvx, cvy], axis=1)
    aug = agg + jnp.dot(rel_feat, rw_ref[:], preferred_element_type=jnp.float32) + rb_ref[:]
    h1 = jnp.maximum(jnp.dot(aug, ow1_ref[:], preferred_element_type=jnp.float32) + ob1_ref[:], 0.0)
    h2 = jnp.maximum(jnp.dot(h1, ow2_ref[:], preferred_element_type=jnp.float32) + ob2_ref[:], 0.0)
    pred = jnp.dot(h2, ow3_ref[:], preferred_element_type=jnp.float32) + ob3_ref[:]

    p0, p1, p2, p3 = pred[:, 0:1], pred[:, 1:2], pred[:, 2:3], pred[:, 3:4]
    g0 = c * p0 - s * p1
    g1 = s * p0 + c * p1
    g2 = c * p2 - s * p3
    g3 = s * p2 + c * p3
    o_ref[0] = x + jnp.concatenate([g0, g1, g2, g3], axis=1)


def kernel(inputs, ef_w1, ef_b1, ef_w2, ef_b2, res1_w, res1_b,
           out_w1, out_b1, out_w2, out_b2, out_w3, out_b3, hidden):
    del hidden
    weights = [ef_w1, ef_b1.reshape(1, H), ef_w2, ef_b2.reshape(1, H),
               res1_w, res1_b.reshape(1, H),
               out_w1, out_b1.reshape(1, H), out_w2, out_b2.reshape(1, H),
               out_w3, out_b3.reshape(1, D_IN)]
    w_specs = [pl.BlockSpec(w.shape, lambda b: (0,) * w.ndim) for w in weights]
    return pl.pallas_call(
        _fused_kernel,
        grid=(B,),
        in_specs=[pl.BlockSpec((1, N, D_IN), lambda b: (b, 0, 0))] + w_specs,
        out_specs=pl.BlockSpec((1, N, D_IN), lambda b: (b, 0, 0)),
        out_shape=jax.ShapeDtypeStruct((B, N, D_IN), jnp.float32),
    )(inputs, *weights)
